# Initial kernel scaffold; baseline (speedup 1.0000x reference)
import jax
import jax.numpy as jnp
from jax.experimental import pallas as pl


def kernel(hidden, feature_bank, expert_bank_idx, ln_gamma, ln_beta, rW1, rb1, rW2, rb2, We1, be1, We2, be2, We3, be3, alpha):
    raise NotImplementedError("write your pallas kernel here")


# speedup vs baseline: 1.4721x; 1.4721x over previous
import functools
import jax
import jax.numpy as jnp
import numpy as np
from jax.experimental import pallas as pl
from jax.experimental.pallas import tpu as pltpu

B = 2048
D = 2048
E = 8
NC = 16
FB = 16
FPE = 2
H = 1024
RH = 1024
K = 2
FD = NC * FB
EF = FPE * FB
RIN = D + FD

M = 256
NT = (B * K) // M + E - 1 + 1
NP = NT * M

INTERPRET = False


def _gelu(x):
    return x * 0.5 * (1.0 + jax.lax.erf(x * np.float32(0.7071067811865476)))


def _router_body(hid_ref, ft_ref, gam_ref, bet_ref, w1_ref, b1_ref, w2_ref,
                 b2_ref, h_ref, g1_ref, g2_ref, i1_ref, i2_ref):
    x = hid_ref[...]
    mu = jnp.mean(x, axis=-1, keepdims=True)
    var = jnp.mean((x - mu) ** 2, axis=-1, keepdims=True)
    h = (x - mu) * jax.lax.rsqrt(var + 1e-5) * gam_ref[...] + bet_ref[...]
    h_ref[...] = h
    r1 = jnp.dot(h, w1_ref[:D], preferred_element_type=jnp.float32)
    r1 = r1 + jnp.dot(ft_ref[...], w1_ref[D:], preferred_element_type=jnp.float32)
    r1 = _gelu(r1 + b1_ref[...])
    logits = jnp.dot(r1, w2_ref[...], preferred_element_type=jnp.float32)
    logits = logits + b2_ref[...]
    ii = jax.lax.broadcasted_iota(jnp.int32, logits.shape, 1)
    v1 = jnp.max(logits, axis=-1, keepdims=True)
    i1 = jnp.min(jnp.where(logits == v1, ii, E), axis=-1, keepdims=True)
    ml = jnp.where(ii == i1, -jnp.inf, logits)
    v2 = jnp.max(ml, axis=-1, keepdims=True)
    i2 = jnp.min(jnp.where(ml == v2, ii, E), axis=-1, keepdims=True)
    e2 = jnp.exp(v2 - v1)
    inv = 1.0 / (1.0 + e2)
    g1_ref[...] = inv
    g2_ref[...] = e2 * inv
    i1_ref[...] = i1
    i2_ref[...] = i2


def _run_router(hidden, feats, ln_gamma, ln_beta, rW1, rb1, rW2, rb2):
    bm = 256
    grid = (B // bm,)
    out_shapes = (
        jax.ShapeDtypeStruct((B, D), jnp.float32),
        jax.ShapeDtypeStruct((B, 1), jnp.float32),
        jax.ShapeDtypeStruct((B, 1), jnp.float32),
        jax.ShapeDtypeStruct((B, 1), jnp.int32),
        jax.ShapeDtypeStruct((B, 1), jnp.int32),
    )
    return pl.pallas_call(
        _router_body,
        grid=grid,
        in_specs=[
            pl.BlockSpec((bm, D), lambda i: (i, 0)),
            pl.BlockSpec((bm, FD), lambda i: (i, 0)),
            pl.BlockSpec((D,), lambda i: (0,)),
            pl.BlockSpec((D,), lambda i: (0,)),
            pl.BlockSpec((RIN, RH), lambda i: (0, 0)),
            pl.BlockSpec((RH,), lambda i: (0,)),
            pl.BlockSpec((RH, E), lambda i: (0, 0)),
            pl.BlockSpec((E,), lambda i: (0,)),
        ],
        out_specs=(
            pl.BlockSpec((bm, D), lambda i: (i, 0)),
            pl.BlockSpec((bm, 1), lambda i: (i, 0)),
            pl.BlockSpec((bm, 1), lambda i: (i, 0)),
            pl.BlockSpec((bm, 1), lambda i: (i, 0)),
            pl.BlockSpec((bm, 1), lambda i: (i, 0)),
        ),
        out_shape=out_shapes,
        interpret=INTERPRET,
    )(hidden, feats, ln_gamma, ln_beta, rW1, rb1, rW2, rb2)


def _expert_body(te_ref, xh_ref, xf_ref, w1h_ref, w1f_ref, b1_ref, w2_ref,
                 b2_ref, w3_ref, b3_ref, y_ref):
    x1 = jnp.dot(xh_ref[...], w1h_ref[0], preferred_element_type=jnp.float32)
    x1 = x1 + jnp.dot(xf_ref[...], w1f_ref[0], preferred_element_type=jnp.float32)
    h1 = _gelu(x1 + b1_ref[0])
    h2 = _gelu(jnp.dot(h1, w2_ref[0], preferred_element_type=jnp.float32) + b2_ref[0])
    y_ref[...] = jnp.dot(h2, w3_ref[0], preferred_element_type=jnp.float32) + b3_ref[0]


def _run_experts(tile_expert, xh, xf, We1h, We1f, be1, We2, be2, We3, be3):
    grid_spec = pltpu.PrefetchScalarGridSpec(
        num_scalar_prefetch=1,
        grid=(NT,),
        in_specs=[
            pl.BlockSpec((M, D), lambda i, te: (i, 0)),
            pl.BlockSpec((M, EF), lambda i, te: (i, 0)),
            pl.BlockSpec((1, D, H), lambda i, te: (te[i], 0, 0)),
            pl.BlockSpec((1, EF, H), lambda i, te: (te[i], 0, 0)),
            pl.BlockSpec((1, 1, H), lambda i, te: (te[i], 0, 0)),
            pl.BlockSpec((1, H, H), lambda i, te: (te[i], 0, 0)),
            pl.BlockSpec((1, 1, H), lambda i, te: (te[i], 0, 0)),
            pl.BlockSpec((1, H, D), lambda i, te: (te[i], 0, 0)),
            pl.BlockSpec((1, 1, D), lambda i, te: (te[i], 0, 0)),
        ],
        out_specs=pl.BlockSpec((M, D), lambda i, te: (i, 0)),
    )
    return pl.pallas_call(
        _expert_body,
        grid_spec=grid_spec,
        out_shape=jax.ShapeDtypeStruct((NP, D), jnp.float32),
        interpret=INTERPRET,
    )(tile_expert, xh, xf, We1h, We1f, be1, We2, be2, We3, be3)


def kernel(hidden, feature_bank, expert_bank_idx, ln_gamma, ln_beta,
           rW1, rb1, rW2, rb2, We1, be1, We2, be2, We3, be3, alpha):
    feats = feature_bank.reshape(B, FD)
    h, g1, g2, i1, i2 = _run_router(
        hidden, feats, ln_gamma, ln_beta, rW1, rb1, rW2, rb2)
    i1 = i1[:, 0]
    i2 = i2[:, 0]

    ids = jnp.concatenate([i1, i2])
    onehot = (ids[:, None] == jnp.arange(E)[None, :]).astype(jnp.int32)
    counts = jnp.sum(onehot, axis=0)
    tiles_e = (counts + M - 1) // M
    tile_start = jnp.concatenate(
        [jnp.zeros((1,), jnp.int32), jnp.cumsum(tiles_e)[:-1]])
    row_start = tile_start * M
    pos = jnp.cumsum(onehot, axis=0) - 1
    pos_a = jnp.take_along_axis(pos, ids[:, None], axis=1)[:, 0]
    slot = row_start[ids] + pos_a
    token = jnp.concatenate([jnp.arange(B, dtype=jnp.int32)] * 2)
    token_of_slot = jnp.zeros((NP,), jnp.int32).at[slot].set(token)
    expert_of_tile = (jnp.searchsorted(
        tile_start, jnp.arange(NT, dtype=jnp.int32), side='right') - 1
                      ).astype(jnp.int32)
    expert_of_slot = jnp.repeat(expert_of_tile, M)
    fidx_of_slot = token_of_slot * E + expert_of_slot

    f2 = feats.reshape(B * E, EF)
    xh = h[token_of_slot]
    xf = f2[fidx_of_slot]

    We1h = We1[:, :D, :]
    We1f = We1[:, D:, :]
    yg = _run_experts(expert_of_tile, xh, xf, We1h, We1f,
                      be1.reshape(E, 1, H), We2, be2.reshape(E, 1, H),
                      We3, be3.reshape(E, 1, D))

    slot1 = slot[:B]
    slot2 = slot[B:]
    comb = g1 * yg[slot1] + g2 * yg[slot2]
    return hidden + alpha * comb
